# Initial kernel scaffold; baseline (speedup 1.0000x reference)
import jax
import jax.numpy as jnp
from jax.experimental import pallas as pl


def kernel(edge_attr, node_emb, vn_emb, conv_W, conv_b, bn_g, bn_b, vn_W1, vn_b1, vn_g1, vn_be1, vn_W2, vn_b2, vn_g2, vn_be2, x, edge_index, batch):
    raise NotImplementedError("write your pallas kernel here")


# speedup vs baseline: 10.0917x; 10.0917x over previous
import functools

import jax
import jax.numpy as jnp
from jax import lax
from jax.experimental import pallas as pl
from jax.experimental.pallas import tpu as pltpu
from jax.experimental.pallas import tpu_sc as plsc

N = 10000
NP = 10240
D = 128
G = 128
E = 320000
NC, NS = 2, 16
NW = NC * NS
EW_PER = E // NW
EPW = 10240
NCH = 80
CH = 128
RT = NP // 128
SROWS = NP // NS
EPS = 1e-5
f32 = jnp.float32
i32 = jnp.int32

_MESH = plsc.VectorSubcoreMesh(
    core_axis_name="c", subcore_axis_name="s", num_cores=NC, num_subcores=NS)


@functools.partial(
    pl.kernel,
    out_type=jax.ShapeDtypeStruct((NC, NP), f32),
    mesh=_MESH,
    scratch_types=[
        pltpu.VMEM((NCH, CH), i32),
        pltpu.VMEM((NCH, CH), f32),
        pltpu.VMEM((SROWS,), f32),
        pltpu.VMEM_SHARED((NP,), f32),
    ],
)
def _deg_sc(dst_hbm, ew_hbm, out_hbm, dst_v, ew_v, zb, deg_sh):
    c = lax.axis_index("c")
    s = lax.axis_index("s")
    w = s * NC + c
    for i in range(SROWS // 16):
        zb[pl.ds(i * 16, 16)] = jnp.zeros((16,), f32)
    pltpu.sync_copy(zb, deg_sh.at[pl.ds(s * SROWS, SROWS)])
    plsc.subcore_barrier()
    pltpu.sync_copy(dst_hbm.at[w], dst_v)
    pltpu.sync_copy(ew_hbm.at[w], ew_v)

    def chunk(j, carry):
        pltpu.sync_copy(ew_v.at[j], deg_sh.at[dst_v.at[j]], add=True)
        return carry

    lax.fori_loop(0, NCH, chunk, 0)
    plsc.subcore_barrier()

    @pl.when(s == 0)
    def _():
        pltpu.sync_copy(deg_sh, out_hbm.at[c])


@functools.partial(
    pl.kernel,
    out_type=jax.ShapeDtypeStruct((NC, NP, D), f32),
    mesh=_MESH,
    scratch_types=[
        pltpu.VMEM((NCH, CH), i32),
        pltpu.VMEM((NCH, CH), i32),
        pltpu.VMEM((EPW,), f32),
        pltpu.VMEM((CH, D), f32),
        pltpu.VMEM_SHARED((NP, D), f32),
        pltpu.SemaphoreType.DMA,
    ],
)
def _msg_sc(hwp_hbm, src_hbm, dst_hbm, ew_hbm, out_hbm,
            src_v, dst_v, ew_v, buf, acc_sh, gsem):
    c = lax.axis_index("c")
    s = lax.axis_index("s")
    w = s * NC + c

    def zrow(i, carry):
        for gblk in range(D // 16):
            buf[i, pl.ds(gblk * 16, 16)] = jnp.zeros((16,), f32)
        return carry

    lax.fori_loop(0, CH, zrow, 0)
    for k in range(SROWS // CH):
        pltpu.sync_copy(buf, acc_sh.at[pl.ds(s * SROWS + k * CH, CH)])
    plsc.subcore_barrier()

    pltpu.sync_copy(src_hbm.at[w], src_v)
    pltpu.sync_copy(dst_hbm.at[w], dst_v)
    pltpu.sync_copy(ew_hbm.at[w], ew_v)

    def chunk(j, carry):
        pltpu.async_copy(hwp_hbm.at[src_v.at[j]], buf, gsem).wait()

        def grp(g, c2):
            ev = ew_v[pl.ds(j * CH + g * 16, 16)]
            for l in range(16):
                ws = jnp.full((16,), ev[l], f32)
                e = g * 16 + l
                for gblk in range(D // 16):
                    sl = pl.ds(gblk * 16, 16)
                    buf[e, sl] = buf[e, sl] * ws
            return c2

        lax.fori_loop(0, CH // 16, grp, 0)
        pltpu.sync_copy(buf, acc_sh.at[dst_v.at[j]], add=True)
        return carry

    lax.fori_loop(0, NCH, chunk, 0)
    plsc.subcore_barrier()

    pltpu.sync_copy(acc_sh.at[pl.ds(s * SROWS, SROWS)],
                    out_hbm.at[c, pl.ds(s * SROWS, SROWS)])


def _dinv_tile(deg_blk):
    return lax.rsqrt(deg_blk[0, 0] + deg_blk[1, 0] + 1.0)


def _graph_onehot(bt):
    kg = lax.broadcasted_iota(i32, (G, 1), 0)
    return (bt == kg).astype(f32)


def _a0_body(x_ref, b_ref, vn_ref, w_ref, emb_ref, deg_ref, hwp_ref, pooled_ref):
    r = pl.program_id(0)
    xrow = x_ref[pl.ds(r, 1), :]
    k8 = lax.broadcasted_iota(i32, (8, 1), 0)
    oh8 = (xrow == k8).astype(f32)
    h = lax.dot_general(oh8, emb_ref[...], (((0,), (0,)), ((), ())),
                        preferred_element_type=f32, precision=lax.Precision.HIGHEST)
    ohg = _graph_onehot(b_ref[pl.ds(r, 1), :])
    vns = lax.dot_general(ohg, vn_ref[...], (((0,), (0,)), ((), ())),
                          preferred_element_type=f32, precision=lax.Precision.HIGHEST)
    h_in = h + vns
    pc = lax.dot_general(ohg, h_in, (((1,), (0,)), ((), ())),
                         preferred_element_type=f32, precision=lax.Precision.HIGHEST)

    @pl.when(r == 0)
    def _():
        pooled_ref[...] = pc

    @pl.when(r > 0)
    def _():
        pooled_ref[...] += pc

    hw = lax.dot_general(h_in, w_ref[...], (((1,), (0,)), ((), ())),
                         preferred_element_type=f32)
    hwp_ref[...] = hw * _dinv_tile(deg_ref[...])


def _a12_body(op_ref, stats_ref, bng_ref, bnb_ref, b_ref, vn_ref, w_ref,
              deg_ref, hwp_ref, pooled_ref):
    r = pl.program_id(0)
    st = stats_ref[...]
    m = st[0:1] * (1.0 / N)
    var = st[1:2] * (1.0 / N) - m * m
    h = jnp.maximum(
        (op_ref[...] - m) * lax.rsqrt(var + EPS) * bng_ref[...] + bnb_ref[...],
        0.0)
    ohg = _graph_onehot(b_ref[pl.ds(r, 1), :])
    vns = lax.dot_general(ohg, vn_ref[...], (((0,), (0,)), ((), ())),
                          preferred_element_type=f32, precision=lax.Precision.HIGHEST)
    h_in = h + vns
    pc = lax.dot_general(ohg, h_in, (((1,), (0,)), ((), ())),
                         preferred_element_type=f32, precision=lax.Precision.HIGHEST)

    @pl.when(r == 0)
    def _():
        pooled_ref[...] = pc

    @pl.when(r > 0)
    def _():
        pooled_ref[...] += pc

    hw = lax.dot_general(h_in, w_ref[...], (((1,), (0,)), ((), ())),
                         preferred_element_type=f32)
    hwp_ref[...] = hw * _dinv_tile(deg_ref[...])


def _post_body(acc0_ref, acc1_ref, hwp_ref, b_ref, deg_ref, op_ref, stats_ref):
    r = pl.program_id(0)
    a = acc0_ref[...] + acc1_ref[...] + hwp_ref[...]
    op = a * _dinv_tile(deg_ref[...]) + b_ref[...]
    op_ref[...] = op
    rowid = lax.broadcasted_iota(i32, (128, 1), 0) + r * 128
    msk = (rowid < N).astype(f32)
    opm = op * msk
    s1 = jnp.sum(opm, axis=0, keepdims=True)
    s2 = jnp.sum(opm * op, axis=0, keepdims=True)
    st = jnp.concatenate([s1, s2], axis=0)

    @pl.when(r == 0)
    def _():
        stats_ref[...] = st

    @pl.when(r > 0)
    def _():
        stats_ref[...] += st


def _vn_body(pooled_ref, vn_ref, w1_ref, b1_ref, g1_ref, be1_ref,
             w2_ref, b2_ref, g2_ref, be2_ref, out_ref):
    pt = pooled_ref[...] + vn_ref[...]
    z = lax.dot_general(pt, w1_ref[...], (((1,), (0,)), ((), ())),
                        preferred_element_type=f32) + b1_ref[...]
    m = jnp.mean(z, axis=0, keepdims=True)
    v = jnp.mean(z * z, axis=0, keepdims=True) - m * m
    z = jnp.maximum((z - m) * lax.rsqrt(v + EPS) * g1_ref[...] + be1_ref[...],
                    0.0)
    z2 = lax.dot_general(z, w2_ref[...], (((1,), (0,)), ((), ())),
                         preferred_element_type=f32) + b2_ref[...]
    m2 = jnp.mean(z2, axis=0, keepdims=True)
    v2 = jnp.mean(z2 * z2, axis=0, keepdims=True) - m2 * m2
    out_ref[...] = jnp.maximum(
        (z2 - m2) * lax.rsqrt(v2 + EPS) * g2_ref[...] + be2_ref[...], 0.0)


def _final_body(op_ref, stats_ref, g_ref, b_ref, out_ref):
    st = stats_ref[...]
    m = st[0:1] * (1.0 / N)
    var = st[1:2] * (1.0 / N) - m * m
    out_ref[...] = ((op_ref[...] - m) * lax.rsqrt(var + EPS) * g_ref[...]
                    + b_ref[...])


def _full(shape):
    nd = len(shape)
    return pl.BlockSpec(shape, lambda r, _nd=nd: (0,) * _nd)


_DEG_SPEC = pl.BlockSpec((2, 1, 128, 1), lambda r: (0, r, 0, 0))
_ROW_SPEC = pl.BlockSpec((128, D), lambda r: (r, 0))


def _a0_call(x2, b2, vn, W, emb, degp4):
    return pl.pallas_call(
        _a0_body,
        grid=(RT,),
        in_specs=[_full((RT, 128)), _full((RT, 128)), _full((G, D)),
                  _full((D, D)), _full((8, D)), _DEG_SPEC],
        out_specs=[_ROW_SPEC, _full((G, D))],
        out_shape=[jax.ShapeDtypeStruct((NP, D), f32),
                   jax.ShapeDtypeStruct((G, D), f32)],
    )(x2, b2, vn, W, emb, degp4)


def _a12_call(op, stats, bng, bnb, b2, vn, W, degp4):
    return pl.pallas_call(
        _a12_body,
        grid=(RT,),
        in_specs=[_ROW_SPEC, _full((2, D)), _full((1, D)), _full((1, D)),
                  _full((RT, 128)), _full((G, D)), _full((D, D)), _DEG_SPEC],
        out_specs=[_ROW_SPEC, _full((G, D))],
        out_shape=[jax.ShapeDtypeStruct((NP, D), f32),
                   jax.ShapeDtypeStruct((G, D), f32)],
    )(op, stats, bng, bnb, b2, vn, W, degp4)


def _post_call(acc0, acc1, hwp, cb, degp4):
    return pl.pallas_call(
        _post_body,
        grid=(RT,),
        in_specs=[_ROW_SPEC, _ROW_SPEC, _ROW_SPEC, _full((1, D)), _DEG_SPEC],
        out_specs=[_ROW_SPEC, _full((2, D))],
        out_shape=[jax.ShapeDtypeStruct((NP, D), f32),
                   jax.ShapeDtypeStruct((2, D), f32)],
    )(acc0, acc1, hwp, cb, degp4)


def _vn_call(pooled, vn, w1, b1, g1, be1, w2, b2, g2, be2):
    return pl.pallas_call(
        _vn_body,
        grid=(1,),
        in_specs=[_full((G, D)), _full((G, D)), _full((D, 2 * D)),
                  _full((1, 2 * D)), _full((1, 2 * D)), _full((1, 2 * D)),
                  _full((2 * D, D)), _full((1, D)), _full((1, D)),
                  _full((1, D))],
        out_specs=_full((G, D)),
        out_shape=jax.ShapeDtypeStruct((G, D), f32),
    )(pooled, vn, w1, b1, g1, be1, w2, b2, g2, be2)


def _final_call(op, stats, g, b):
    return pl.pallas_call(
        _final_body,
        grid=(RT,),
        in_specs=[_ROW_SPEC, _full((2, D)), _full((1, D)), _full((1, D))],
        out_specs=_ROW_SPEC,
        out_shape=jax.ShapeDtypeStruct((NP, D), f32),
    )(op, stats, g, b)


def kernel(edge_attr, node_emb, vn_emb, conv_W, conv_b, bn_g, bn_b,
           vn_W1, vn_b1, vn_g1, vn_be1, vn_W2, vn_b2, vn_g2, vn_be2,
           x, edge_index, batch):
    src = edge_index[0].astype(i32)
    dst = edge_index[1].astype(i32)
    ew = edge_attr.astype(f32)

    padw = EPW - EW_PER
    wid = jnp.arange(NW, dtype=i32)[:, None]
    padv = (wid * 131 + jnp.arange(padw, dtype=i32)[None, :] * 37) % N
    src3 = jnp.concatenate([src.reshape(NW, EW_PER), padv],
                           axis=1).reshape(NW, NCH, CH)
    dst3 = jnp.concatenate([dst.reshape(NW, EW_PER), padv],
                           axis=1).reshape(NW, NCH, CH)
    ew2 = jnp.concatenate([ew.reshape(NW, EW_PER),
                           jnp.zeros((NW, padw), f32)], axis=1)
    ew3 = ew2.reshape(NW, NCH, CH)

    x2 = jnp.pad(x.astype(i32), (0, NP - N)).reshape(RT, 128)
    b2 = jnp.pad(batch.astype(i32), (0, NP - N),
                 constant_values=1 << 20).reshape(RT, 128)
    vn = jnp.broadcast_to(vn_emb[0], (G, D)).astype(f32)

    deg_p = _deg_sc(dst3, ew3)
    degp4 = deg_p.reshape(2, RT, 128, 1)

    op = stats = None
    for l in range(3):
        if l == 0:
            hwp, pooled = _a0_call(x2, b2, vn, conv_W[0], node_emb, degp4)
        else:
            hwp, pooled = _a12_call(op, stats, bn_g[l - 1].reshape(1, D),
                                    bn_b[l - 1].reshape(1, D), b2, vn,
                                    conv_W[l], degp4)
        acc = _msg_sc(hwp, src3, dst3, ew2)
        op, stats = _post_call(acc[0], acc[1], hwp,
                               conv_b[l].reshape(1, D), degp4)
        if l < 2:
            vn = _vn_call(pooled, vn,
                          vn_W1[l], vn_b1[l].reshape(1, 2 * D),
                          vn_g1[l].reshape(1, 2 * D),
                          vn_be1[l].reshape(1, 2 * D),
                          vn_W2[l], vn_b2[l].reshape(1, D),
                          vn_g2[l].reshape(1, D), vn_be2[l].reshape(1, D))

    h = _final_call(op, stats, bn_g[2].reshape(1, D), bn_b[2].reshape(1, D))
    return h[:N]


# speedup vs baseline: 12.6777x; 1.2563x over previous
import functools

import jax
import jax.numpy as jnp
from jax import lax
from jax.experimental import pallas as pl
from jax.experimental.pallas import tpu as pltpu
from jax.experimental.pallas import tpu_sc as plsc

N = 10000
NP = 10240
D = 128
G = 128
E = 320000
NC, NS = 2, 16
NW = NC * NS
EW_PER = E // NW
EPW = 10240
NCH = 80
CH = 128
RT = NP // 128
SROWS = NP // NS
EPS = 1e-5
f32 = jnp.float32
i32 = jnp.int32

_MESH = plsc.VectorSubcoreMesh(
    core_axis_name="c", subcore_axis_name="s", num_cores=NC, num_subcores=NS)


@functools.partial(
    pl.kernel,
    out_type=jax.ShapeDtypeStruct((NC, NP), f32),
    mesh=_MESH,
    scratch_types=[
        pltpu.VMEM((NCH, CH), i32),
        pltpu.VMEM((NCH, CH), f32),
        pltpu.VMEM((SROWS,), f32),
        pltpu.VMEM_SHARED((NP,), f32),
    ],
)
def _deg_sc(dst_hbm, ew_hbm, out_hbm, dst_v, ew_v, zb, deg_sh):
    c = lax.axis_index("c")
    s = lax.axis_index("s")
    w = s * NC + c
    for i in range(SROWS // 16):
        zb[pl.ds(i * 16, 16)] = jnp.zeros((16,), f32)
    pltpu.sync_copy(zb, deg_sh.at[pl.ds(s * SROWS, SROWS)])
    plsc.subcore_barrier()
    pltpu.sync_copy(dst_hbm.at[w], dst_v)
    pltpu.sync_copy(ew_hbm.at[w], ew_v)

    def chunk(j, carry):
        pltpu.sync_copy(ew_v.at[j], deg_sh.at[dst_v.at[j]], add=True)
        return carry

    lax.fori_loop(0, NCH, chunk, 0)
    plsc.subcore_barrier()

    @pl.when(s == 0)
    def _():
        pltpu.sync_copy(deg_sh, out_hbm.at[c])


def _scale_chunk(buf, ew_v, ci):
    def grp(g, c2):
        ev = ew_v[pl.ds(ci * CH + g * 16, 16)]
        for l in range(16):
            ws = jnp.full((16,), ev[l], f32)
            e = g * 16 + l
            for gblk in range(D // 16):
                sl = pl.ds(gblk * 16, 16)
                buf[e, sl] = buf[e, sl] * ws
        return c2

    lax.fori_loop(0, CH // 16, grp, 0)


SSC = 16
NSC = NCH // SSC


@functools.partial(
    pl.kernel,
    out_type=jax.ShapeDtypeStruct((NC, NP, D), f32),
    mesh=_MESH,
    scratch_types=[
        pltpu.VMEM((SSC, CH), i32),
        pltpu.VMEM((SSC, CH), i32),
        pltpu.VMEM((SSC * CH,), f32),
        pltpu.VMEM((CH, D), f32),
        pltpu.VMEM((CH, D), f32),
        pltpu.VMEM_SHARED((NP, D), f32),
        pltpu.SemaphoreType.DMA,
        pltpu.SemaphoreType.DMA,
    ],
)
def _msg_sc(hwp_hbm, src_hbm, dst_hbm, ew_hbm, out_hbm,
            src_v, dst_v, ew_v, buf0, buf1, acc_sh, g0, g1):
    c = lax.axis_index("c")
    s = lax.axis_index("s")
    w = s * NC + c

    def zrow(i, carry):
        for gblk in range(D // 16):
            buf0[i, pl.ds(gblk * 16, 16)] = jnp.zeros((16,), f32)
        return carry

    lax.fori_loop(0, CH, zrow, 0)
    for k in range(SROWS // CH):
        pltpu.sync_copy(buf0, acc_sh.at[pl.ds(s * SROWS + k * CH, CH)])
    plsc.subcore_barrier()

    def superchunk(sc, carry):
        pltpu.sync_copy(src_hbm.at[w, pl.ds(sc * SSC, SSC)], src_v)
        pltpu.sync_copy(dst_hbm.at[w, pl.ds(sc * SSC, SSC)], dst_v)
        pltpu.sync_copy(ew_hbm.at[w, pl.ds(sc * SSC * CH, SSC * CH)], ew_v)
        pltpu.async_copy(hwp_hbm.at[src_v.at[0]], buf0, g0)
        pltpu.async_copy(hwp_hbm.at[src_v.at[1]], buf1, g1)

        def pair(k, c2):
            for i, (bi, gi) in enumerate(((buf0, g0), (buf1, g1))):
                ci = 2 * k + i
                pltpu.make_async_copy(hwp_hbm.at[src_v.at[ci]], bi, gi).wait()
                _scale_chunk(bi, ew_v, ci)
                pltpu.sync_copy(bi, acc_sh.at[dst_v.at[ci]], add=True)

                @pl.when(ci + 2 < SSC)
                def _():
                    pltpu.async_copy(hwp_hbm.at[src_v.at[ci + 2]], bi, gi)
            return c2

        lax.fori_loop(0, SSC // 2, pair, 0)
        return carry

    lax.fori_loop(0, NSC, superchunk, 0)
    plsc.subcore_barrier()

    pltpu.sync_copy(acc_sh.at[pl.ds(s * SROWS, SROWS)],
                    out_hbm.at[c, pl.ds(s * SROWS, SROWS)])


def _dinv_tile(deg_blk):
    return lax.rsqrt(deg_blk[0, 0] + deg_blk[1, 0] + 1.0)


def _graph_onehot(bt):
    kg = lax.broadcasted_iota(i32, (G, 1), 0)
    return (bt == kg).astype(f32)


def _a0_body(x_ref, b_ref, vn_ref, w_ref, emb_ref, deg_ref, hwp_ref, pooled_ref):
    r = pl.program_id(0)
    xrow = x_ref[pl.ds(r, 1), :]
    k8 = lax.broadcasted_iota(i32, (8, 1), 0)
    oh8 = (xrow == k8).astype(f32)
    h = lax.dot_general(oh8, emb_ref[...], (((0,), (0,)), ((), ())),
                        preferred_element_type=f32, precision=lax.Precision.HIGHEST)
    ohg = _graph_onehot(b_ref[pl.ds(r, 1), :])
    vns = lax.dot_general(ohg, vn_ref[...], (((0,), (0,)), ((), ())),
                          preferred_element_type=f32, precision=lax.Precision.HIGHEST)
    h_in = h + vns
    pc = lax.dot_general(ohg, h_in, (((1,), (0,)), ((), ())),
                         preferred_element_type=f32, precision=lax.Precision.HIGHEST)

    @pl.when(r == 0)
    def _():
        pooled_ref[...] = pc

    @pl.when(r > 0)
    def _():
        pooled_ref[...] += pc

    hw = lax.dot_general(h_in, w_ref[...], (((1,), (0,)), ((), ())),
                         preferred_element_type=f32)
    hwp_ref[...] = hw * _dinv_tile(deg_ref[...])


def _a12_body(op_ref, stats_ref, bng_ref, bnb_ref, b_ref, vn_ref, w_ref,
              deg_ref, hwp_ref, pooled_ref):
    r = pl.program_id(0)
    st = stats_ref[...]
    m = st[0:1] * (1.0 / N)
    var = st[1:2] * (1.0 / N) - m * m
    h = jnp.maximum(
        (op_ref[...] - m) * lax.rsqrt(var + EPS) * bng_ref[...] + bnb_ref[...],
        0.0)
    ohg = _graph_onehot(b_ref[pl.ds(r, 1), :])
    vns = lax.dot_general(ohg, vn_ref[...], (((0,), (0,)), ((), ())),
                          preferred_element_type=f32, precision=lax.Precision.HIGHEST)
    h_in = h + vns
    pc = lax.dot_general(ohg, h_in, (((1,), (0,)), ((), ())),
                         preferred_element_type=f32, precision=lax.Precision.HIGHEST)

    @pl.when(r == 0)
    def _():
        pooled_ref[...] = pc

    @pl.when(r > 0)
    def _():
        pooled_ref[...] += pc

    hw = lax.dot_general(h_in, w_ref[...], (((1,), (0,)), ((), ())),
                         preferred_element_type=f32)
    hwp_ref[...] = hw * _dinv_tile(deg_ref[...])


def _post_body(acc0_ref, acc1_ref, hwp_ref, b_ref, deg_ref, op_ref, stats_ref):
    r = pl.program_id(0)
    a = acc0_ref[...] + acc1_ref[...] + hwp_ref[...]
    op = a * _dinv_tile(deg_ref[...]) + b_ref[...]
    op_ref[...] = op
    rowid = lax.broadcasted_iota(i32, (128, 1), 0) + r * 128
    msk = (rowid < N).astype(f32)
    opm = op * msk
    s1 = jnp.sum(opm, axis=0, keepdims=True)
    s2 = jnp.sum(opm * op, axis=0, keepdims=True)
    st = jnp.concatenate([s1, s2], axis=0)

    @pl.when(r == 0)
    def _():
        stats_ref[...] = st

    @pl.when(r > 0)
    def _():
        stats_ref[...] += st


def _vn_body(pooled_ref, vn_ref, w1_ref, b1_ref, g1_ref, be1_ref,
             w2_ref, b2_ref, g2_ref, be2_ref, out_ref):
    pt = pooled_ref[...] + vn_ref[...]
    z = lax.dot_general(pt, w1_ref[...], (((1,), (0,)), ((), ())),
                        preferred_element_type=f32) + b1_ref[...]
    m = jnp.mean(z, axis=0, keepdims=True)
    v = jnp.mean(z * z, axis=0, keepdims=True) - m * m
    z = jnp.maximum((z - m) * lax.rsqrt(v + EPS) * g1_ref[...] + be1_ref[...],
                    0.0)
    z2 = lax.dot_general(z, w2_ref[...], (((1,), (0,)), ((), ())),
                         preferred_element_type=f32) + b2_ref[...]
    m2 = jnp.mean(z2, axis=0, keepdims=True)
    v2 = jnp.mean(z2 * z2, axis=0, keepdims=True) - m2 * m2
    out_ref[...] = jnp.maximum(
        (z2 - m2) * lax.rsqrt(v2 + EPS) * g2_ref[...] + be2_ref[...], 0.0)


def _final_body(op_ref, stats_ref, g_ref, b_ref, out_ref):
    st = stats_ref[...]
    m = st[0:1] * (1.0 / N)
    var = st[1:2] * (1.0 / N) - m * m
    out_ref[...] = ((op_ref[...] - m) * lax.rsqrt(var + EPS) * g_ref[...]
                    + b_ref[...])


def _full(shape):
    nd = len(shape)
    return pl.BlockSpec(shape, lambda r, _nd=nd: (0,) * _nd)


_DEG_SPEC = pl.BlockSpec((2, 1, 128, 1), lambda r: (0, r, 0, 0))
_ROW_SPEC = pl.BlockSpec((128, D), lambda r: (r, 0))


def _a0_call(x2, b2, vn, W, emb, degp4):
    return pl.pallas_call(
        _a0_body,
        grid=(RT,),
        in_specs=[_full((RT, 128)), _full((RT, 128)), _full((G, D)),
                  _full((D, D)), _full((8, D)), _DEG_SPEC],
        out_specs=[_ROW_SPEC, _full((G, D))],
        out_shape=[jax.ShapeDtypeStruct((NP, D), f32),
                   jax.ShapeDtypeStruct((G, D), f32)],
    )(x2, b2, vn, W, emb, degp4)


def _a12_call(op, stats, bng, bnb, b2, vn, W, degp4):
    return pl.pallas_call(
        _a12_body,
        grid=(RT,),
        in_specs=[_ROW_SPEC, _full((2, D)), _full((1, D)), _full((1, D)),
                  _full((RT, 128)), _full((G, D)), _full((D, D)), _DEG_SPEC],
        out_specs=[_ROW_SPEC, _full((G, D))],
        out_shape=[jax.ShapeDtypeStruct((NP, D), f32),
                   jax.ShapeDtypeStruct((G, D), f32)],
    )(op, stats, bng, bnb, b2, vn, W, degp4)


def _post_call(acc0, acc1, hwp, cb, degp4):
    return pl.pallas_call(
        _post_body,
        grid=(RT,),
        in_specs=[_ROW_SPEC, _ROW_SPEC, _ROW_SPEC, _full((1, D)), _DEG_SPEC],
        out_specs=[_ROW_SPEC, _full((2, D))],
        out_shape=[jax.ShapeDtypeStruct((NP, D), f32),
                   jax.ShapeDtypeStruct((2, D), f32)],
    )(acc0, acc1, hwp, cb, degp4)


def _vn_call(pooled, vn, w1, b1, g1, be1, w2, b2, g2, be2):
    return pl.pallas_call(
        _vn_body,
        grid=(1,),
        in_specs=[_full((G, D)), _full((G, D)), _full((D, 2 * D)),
                  _full((1, 2 * D)), _full((1, 2 * D)), _full((1, 2 * D)),
                  _full((2 * D, D)), _full((1, D)), _full((1, D)),
                  _full((1, D))],
        out_specs=_full((G, D)),
        out_shape=jax.ShapeDtypeStruct((G, D), f32),
    )(pooled, vn, w1, b1, g1, be1, w2, b2, g2, be2)


def _final_call(op, stats, g, b):
    return pl.pallas_call(
        _final_body,
        grid=(RT,),
        in_specs=[_ROW_SPEC, _full((2, D)), _full((1, D)), _full((1, D))],
        out_specs=_ROW_SPEC,
        out_shape=jax.ShapeDtypeStruct((NP, D), f32),
    )(op, stats, g, b)


def kernel(edge_attr, node_emb, vn_emb, conv_W, conv_b, bn_g, bn_b,
           vn_W1, vn_b1, vn_g1, vn_be1, vn_W2, vn_b2, vn_g2, vn_be2,
           x, edge_index, batch):
    src = edge_index[0].astype(i32)
    dst = edge_index[1].astype(i32)
    ew = edge_attr.astype(f32)

    padw = EPW - EW_PER
    wid = jnp.arange(NW, dtype=i32)[:, None]
    padv = (wid * 131 + jnp.arange(padw, dtype=i32)[None, :] * 37) % N
    src3 = jnp.concatenate([src.reshape(NW, EW_PER), padv],
                           axis=1).reshape(NW, NCH, CH)
    dst3 = jnp.concatenate([dst.reshape(NW, EW_PER), padv],
                           axis=1).reshape(NW, NCH, CH)
    ew2 = jnp.concatenate([ew.reshape(NW, EW_PER),
                           jnp.zeros((NW, padw), f32)], axis=1)
    ew3 = ew2.reshape(NW, NCH, CH)

    x2 = jnp.pad(x.astype(i32), (0, NP - N)).reshape(RT, 128)
    b2 = jnp.pad(batch.astype(i32), (0, NP - N),
                 constant_values=1 << 20).reshape(RT, 128)
    vn = jnp.broadcast_to(vn_emb[0], (G, D)).astype(f32)

    deg_p = _deg_sc(dst3, ew3)
    degp4 = deg_p.reshape(2, RT, 128, 1)

    op = stats = None
    for l in range(3):
        if l == 0:
            hwp, pooled = _a0_call(x2, b2, vn, conv_W[0], node_emb, degp4)
        else:
            hwp, pooled = _a12_call(op, stats, bn_g[l - 1].reshape(1, D),
                                    bn_b[l - 1].reshape(1, D), b2, vn,
                                    conv_W[l], degp4)
        acc = _msg_sc(hwp, src3, dst3, ew2)
        op, stats = _post_call(acc[0], acc[1], hwp,
                               conv_b[l].reshape(1, D), degp4)
        if l < 2:
            vn = _vn_call(pooled, vn,
                          vn_W1[l], vn_b1[l].reshape(1, 2 * D),
                          vn_g1[l].reshape(1, 2 * D),
                          vn_be1[l].reshape(1, 2 * D),
                          vn_W2[l], vn_b2[l].reshape(1, D),
                          vn_g2[l].reshape(1, D), vn_be2[l].reshape(1, D))

    h = _final_call(op, stats, bn_g[2].reshape(1, D), bn_b[2].reshape(1, D))
    return h[:N]
